# Initial kernel scaffold; baseline (speedup 1.0000x reference)
import jax
import jax.numpy as jnp
from jax.experimental import pallas as pl


def kernel(coordinates, W_emb, b_emb, enc_Wq, enc_Wk, enc_Wv, enc_Wo, enc_ln1_g, enc_ln1_b, enc_W1, enc_b1, enc_W2, enc_b2, enc_ln2_g, enc_ln2_b, dec_Wq, dec_Wk, dec_Wv, dec_Wo, dec_Wp):
    raise NotImplementedError("write your pallas kernel here")


# speedup vs baseline: 1.6037x; 1.6037x over previous
import functools
import math

import jax
import jax.numpy as jnp
from jax import lax
from jax.experimental import pallas as pl

H = 8
NEG = -1e9


def _bfu(x):
    return x.astype(jnp.bfloat16).astype(jnp.float32)


def _mm(a, b, dn):
    return lax.dot_general(a.astype(jnp.bfloat16), b.astype(jnp.bfloat16), dn,
                           preferred_element_type=jnp.float32)


def _stable_softmax(x, axis):
    m = jnp.max(x, axis=axis, keepdims=True)
    e = jnp.exp(x - m)
    return e / jnp.sum(e, axis=axis, keepdims=True)


def _layernorm(x, g, b):
    m = jnp.mean(x, axis=-1, keepdims=True)
    c = x - m
    v = jnp.mean(c * c, axis=-1, keepdims=True)
    return c / jnp.sqrt(v + 1e-5) * g + b


def _enc_kernel(L, coords_ref, Wemb_ref, bemb_ref, Wqh_ref, Wkh_ref, Wvh_ref,
                Woh_ref, g1_ref, bb1_ref, W1_ref, b1_ref, W2_ref, b2_ref,
                g2_ref, bb2_ref, out_ref):
    coords = coords_ref[...]
    Wemb = Wemb_ref[...]
    D = Wemb.shape[1]
    x = _bfu(coords[:, :, 0:1] * Wemb[0].reshape(1, 1, D)
             + coords[:, :, 1:2] * Wemb[1].reshape(1, 1, D)
             + bemb_ref[...].reshape(1, 1, D))
    dn_proj = (((2,), (0,)), ((), ()))
    dn_qkT = (((2,), (2,)), ((0,), (0,)))
    dn_av = (((2,), (1,)), ((0,), (0,)))
    for l in range(L):
        acc = None
        for h in range(H):
            q = _bfu(_mm(x, Wqh_ref[l, h], dn_proj))
            k = _bfu(_mm(x, Wkh_ref[l, h], dn_proj))
            v = _bfu(_mm(x, Wvh_ref[l, h], dn_proj))
            s = _mm(q, k, dn_qkT) / 4.0
            att = _stable_softmax(s, axis=-1)
            o = _bfu(_mm(att, v, dn_av))
            contrib = _mm(o, Woh_ref[l, h], dn_proj)
            acc = contrib if acc is None else acc + contrib
        x = _layernorm(x + acc, g1_ref[l], bb1_ref[l])
        hdn = _mm(x, W1_ref[l], dn_proj) + b1_ref[l]
        f = _mm(jnp.maximum(hdn, 0.0), W2_ref[l], dn_proj) + b2_ref[l]
        x = _bfu(_layernorm(x + f, g2_ref[l], bb2_ref[l]))
    out_ref[...] = x


def _dec_kernel(T, enc_ref, G_ref, Wqf_ref, Wqc_ref, Wk_ref, Wv_ref, Wo_ref,
                Wp_ref, Blk_ref, tour_ref, lp_ref):
    enc = enc_ref[...]
    Bb, S, D = enc.shape
    dn_proj = (((2,), (0,)), ((), ()))
    K = _mm(enc, Wk_ref[...], dn_proj)
    V = _mm(enc, Wv_ref[...], dn_proj)
    P = _bfu(_mm(enc, Wp_ref[...], dn_proj))
    Qc = _mm(enc, Wqc_ref[...], dn_proj)
    first_q = _mm(enc[:, 0, :], Wqf_ref[...], (((1,), (0,)), ((), ())))
    Blk = Blk_ref[...]
    Wo = Wo_ref[...]
    sqrt_d = math.sqrt(float(D))

    iota_s = lax.broadcasted_iota(jnp.int32, (Bb, S), 1)
    iota_t = lax.broadcasted_iota(jnp.int32, (Bb, T), 1)
    mask0 = jnp.where(iota_s == 0, 0.0, 1.0)
    tour0 = jnp.zeros((Bb, S), jnp.int32)
    lp0 = jnp.zeros((Bb, T), jnp.float32)
    curq0 = Qc[:, 0, :]

    def body(t, carry):
        mask, cur_q, tourmat, lpmat = carry
        g = G_ref[pl.ds(t, 1)].reshape(Bb, S)
        q = first_q + cur_q
        prod = q[:, None, :] * K
        lgx = lax.dot_general(prod, Blk, dn_proj,
                              precision=lax.Precision.HIGHEST) / 4.0
        lgx = jnp.where(mask[:, :, None] > 0, lgx, NEG)
        attx = _stable_softmax(lgx, axis=1)
        gl = _bfu(jnp.sum(attx * V, axis=1))
        glo = _bfu(_mm(gl, Wo, (((1,), (0,)), ((), ()))))
        u = jnp.sum(glo[:, None, :] * P, axis=2) / sqrt_d
        u = jnp.where(mask > 0, 10.0 * jnp.tanh(u), NEG)
        probs = _stable_softmax(u, axis=-1)
        y = jnp.log(probs + 1e-20)
        z = y + g
        zm = jnp.max(z, axis=-1, keepdims=True)
        sel = jnp.min(jnp.where(z == zm, iota_s, S), axis=-1, keepdims=True)
        ohf = jnp.where(iota_s == sel, 1.0, 0.0)
        lpv = jnp.sum(y * ohf, axis=-1, keepdims=True)
        mask = mask * (1.0 - ohf)
        cur_q = jnp.sum(Qc * ohf[:, :, None], axis=1)
        tourmat = jnp.where(iota_s == t + 1, sel, tourmat)
        lpmat = jnp.where(iota_t == t, lpv, lpmat)
        return mask, cur_q, tourmat, lpmat

    _, _, tourmat, lpmat = lax.fori_loop(0, T, body, (mask0, curq0, tour0, lp0))
    tour_ref[0] = tourmat
    lp_ref[0] = lpmat


def kernel(coordinates, W_emb, b_emb, enc_Wq, enc_Wk, enc_Wv, enc_Wo,
           enc_ln1_g, enc_ln1_b, enc_W1, enc_b1, enc_W2, enc_b2,
           enc_ln2_g, enc_ln2_b, dec_Wq, dec_Wk, dec_Wv, dec_Wo, dec_Wp):
    B, S, IN = coordinates.shape
    D = W_emb.shape[1]
    L = enc_Wq.shape[0]
    DFF = enc_W1.shape[2]
    dh = D // H
    T = S - 1

    BBE = 32
    Wqh = enc_Wq.reshape(L, D, H, dh).transpose(0, 2, 1, 3)
    Wkh = enc_Wk.reshape(L, D, H, dh).transpose(0, 2, 1, 3)
    Wvh = enc_Wv.reshape(L, D, H, dh).transpose(0, 2, 1, 3)
    Woh = enc_Wo.reshape(L, H, dh, D)
    full = lambda a: pl.BlockSpec(a.shape, lambda i: (0,) * a.ndim)
    enc_out = pl.pallas_call(
        functools.partial(_enc_kernel, L),
        grid=(B // BBE,),
        in_specs=[
            pl.BlockSpec((BBE, S, IN), lambda i: (i, 0, 0)),
            full(W_emb), full(b_emb.reshape(1, D)),
            full(Wqh), full(Wkh), full(Wvh), full(Woh),
            full(enc_ln1_g), full(enc_ln1_b),
            full(enc_W1), full(enc_b1.reshape(L, 1, DFF)),
            full(enc_W2), full(enc_b2.reshape(L, 1, D)),
            full(enc_ln2_g), full(enc_ln2_b),
        ],
        out_specs=pl.BlockSpec((BBE, S, D), lambda i: (i, 0, 0)),
        out_shape=jax.ShapeDtypeStruct((B, S, D), jnp.float32),
    )(coordinates, W_emb, b_emb.reshape(1, D), Wqh, Wkh, Wvh, Woh,
      enc_ln1_g, enc_ln1_b, enc_W1, enc_b1.reshape(L, 1, DFF),
      enc_W2, enc_b2.reshape(L, 1, D), enc_ln2_g, enc_ln2_b)

    skey = jax.random.key(1)
    keys = jax.vmap(lambda t: jax.random.fold_in(skey, t))(jnp.arange(T))
    G = jax.vmap(lambda k: jax.random.gumbel(k, (B, S), jnp.float32))(keys)

    NBD = 2
    BBD = B // NBD
    eye = jnp.eye(H, dtype=jnp.float32)
    Blk = jnp.repeat(jnp.repeat(eye, dh, axis=0), dh, axis=1)
    Wqf, Wqc = dec_Wq[:D], dec_Wq[D:]
    tour2, lp2 = pl.pallas_call(
        functools.partial(_dec_kernel, T),
        grid=(NBD,),
        in_specs=[
            pl.BlockSpec((BBD, S, D), lambda i: (i, 0, 0)),
            pl.BlockSpec((T, BBD, S), lambda i: (0, i, 0)),
            full(Wqf), full(Wqc), full(dec_Wk), full(dec_Wv),
            full(dec_Wo), full(dec_Wp), full(Blk),
        ],
        out_specs=[
            pl.BlockSpec((1, BBD, S), lambda i: (i, 0, 0)),
            pl.BlockSpec((1, BBD, T), lambda i: (i, 0, 0)),
        ],
        out_shape=[
            jax.ShapeDtypeStruct((NBD, BBD, S), jnp.int32),
            jax.ShapeDtypeStruct((NBD, BBD, T), jnp.float32),
        ],
    )(enc_out, G, Wqf, Wqc, dec_Wk, dec_Wv, dec_Wo, dec_Wp, Blk)
    return tour2.reshape(B, S), lp2.reshape(B, T)
